# baseline (speedup 1.0000x reference)
import functools

import jax
import jax.numpy as jnp
from jax.experimental import pallas as pl
from jax.experimental.pallas import tpu as pltpu

N_TOK = 2048
D = 1024
H = 4096
E = 10
KTOP = 4
EPAD = 128
NEG = -1e30

TOK_BLK = 256
H_BLK = 2048
NI = N_TOK // TOK_BLK
NHC = H // H_BLK


def _routing_body(x_ref, wg_ref, gates_ref, loss_ref):
    logits = jnp.dot(x_ref[...], wg_ref[...], preferred_element_type=jnp.float32)
    lane = jax.lax.broadcasted_iota(jnp.int32, (N_TOK, EPAD), 1)
    logits = jnp.where(lane < E, logits, NEG)

    rem = logits
    idxs = []
    vals = []
    for _ in range(KTOP):
        m = jnp.max(rem, axis=1, keepdims=True)
        sel_idx = jnp.min(jnp.where(rem == m, lane, EPAD), axis=1, keepdims=True)
        idxs.append(sel_idx)
        vals.append(m)
        rem = jnp.where(lane == sel_idx, NEG, rem)

    exps = [jnp.exp(v - vals[0]) for v in vals]
    denom = exps[0] + exps[1] + exps[2] + exps[3]
    gates = jnp.zeros((N_TOK, EPAD), jnp.float32)
    for k in range(KTOP):
        gates = gates + jnp.where(lane == idxs[k], exps[k] / denom, 0.0)
    gates_ref[...] = gates

    emask = lane[0:1, :] < E
    importance = jnp.sum(gates, axis=0, keepdims=True)
    load = jnp.sum((gates > 0.0).astype(jnp.float32), axis=0, keepdims=True)

    def cv_sq(v):
        s = jnp.sum(jnp.where(emask, v, 0.0))
        mean = s / E
        var = jnp.sum(jnp.where(emask, (v - mean) ** 2, 0.0)) / (E - 1)
        return var / (mean * mean + 1e-10)

    loss_ref[0, 0] = (cv_sq(importance) + cv_sq(load)) * 1e-2


def _ffn_body(x_ref, gates_ref, fc1w_ref, fc1b_ref, fc2w_ref, fc2b_ref,
              y_ref, oe_acc, y_acc):
    e = pl.program_id(1)
    hc = pl.program_id(2)

    h = jnp.dot(x_ref[...], fc1w_ref[0], preferred_element_type=jnp.float32)
    h = jnp.maximum(h + fc1b_ref[0], 0.0)
    part = jnp.dot(h, fc2w_ref[0], preferred_element_type=jnp.float32)

    @pl.when(hc == 0)
    def _():
        oe_acc[...] = part + fc2b_ref[0]

    @pl.when(hc > 0)
    def _():
        oe_acc[...] += part

    @pl.when(hc == NHC - 1)
    def _():
        oe = oe_acc[...]
        m = jnp.max(oe, axis=1, keepdims=True)
        p = jnp.exp(oe - m)
        sm = p / jnp.sum(p, axis=1, keepdims=True)
        lane = jax.lax.broadcasted_iota(jnp.int32, (TOK_BLK, EPAD), 1)
        g = jnp.sum(jnp.where(lane == e, gates_ref[...], 0.0), axis=1,
                    keepdims=True)
        contrib = g * sm

        @pl.when(e == 0)
        def _():
            y_acc[...] = contrib

        @pl.when(e > 0)
        def _():
            y_acc[...] += contrib

        @pl.when(e == E - 1)
        def _():
            y_ref[...] = y_acc[...]


@jax.jit
def kernel(x, w_gate, fc1_w, fc1_b, fc2_w, fc2_b):
    b, l, d = x.shape
    xf = x.reshape(l, d)
    wg = jnp.zeros((D, EPAD), jnp.float32).at[:, :E].set(w_gate)

    gates, loss = pl.pallas_call(
        _routing_body,
        out_shape=(
            jax.ShapeDtypeStruct((N_TOK, EPAD), jnp.float32),
            jax.ShapeDtypeStruct((1, 1), jnp.float32),
        ),
        in_specs=[
            pl.BlockSpec((N_TOK, D), lambda: (0, 0)),
            pl.BlockSpec((D, EPAD), lambda: (0, 0)),
        ],
        out_specs=(
            pl.BlockSpec((N_TOK, EPAD), lambda: (0, 0)),
            pl.BlockSpec((1, 1), lambda: (0, 0), memory_space=pltpu.SMEM),
        ),
    )(xf, wg)

    y = pl.pallas_call(
        _ffn_body,
        grid=(NI, E, NHC),
        in_specs=[
            pl.BlockSpec((TOK_BLK, D), lambda i, e, hc: (i, 0)),
            pl.BlockSpec((TOK_BLK, EPAD), lambda i, e, hc: (i, 0)),
            pl.BlockSpec((1, D, H_BLK), lambda i, e, hc: (e, 0, hc)),
            pl.BlockSpec((1, 1, H_BLK), lambda i, e, hc: (e, 0, hc)),
            pl.BlockSpec((1, H_BLK, D), lambda i, e, hc: (e, hc, 0)),
            pl.BlockSpec((1, 1, D), lambda i, e, hc: (e, 0, 0)),
        ],
        out_specs=pl.BlockSpec((TOK_BLK, D), lambda i, e, hc: (i, 0)),
        out_shape=jax.ShapeDtypeStruct((N_TOK, D), jnp.float32),
        scratch_shapes=[
            pltpu.VMEM((TOK_BLK, D), jnp.float32),
            pltpu.VMEM((TOK_BLK, D), jnp.float32),
        ],
        compiler_params=pltpu.CompilerParams(
            dimension_semantics=("arbitrary", "arbitrary", "arbitrary"),
        ),
    )(xf, gates, fc1_w, fc1_b.reshape(E, 1, H), fc2_w, fc2_b.reshape(E, 1, D))

    return y.reshape(b, l, d), loss.reshape(())


# speedup vs baseline: 1.1785x; 1.1785x over previous
import functools

import jax
import jax.numpy as jnp
from jax.experimental import pallas as pl
from jax.experimental.pallas import tpu as pltpu

N_TOK = 2048
D = 1024
H = 4096
E = 10
KTOP = 4
EPAD = 128
NEG = -1e30

TOK_BLK = 256
H_BLK = 2048
NI = N_TOK // TOK_BLK
NHC = H // H_BLK


def _routing_body(x_ref, wg_ref, gates_ref, loss_ref):
    logits = jnp.dot(x_ref[...], wg_ref[...], preferred_element_type=jnp.float32)
    lane = jax.lax.broadcasted_iota(jnp.int32, (N_TOK, EPAD), 1)
    logits = jnp.where(lane < E, logits, NEG)

    rem = logits
    idxs = []
    vals = []
    for _ in range(KTOP):
        m = jnp.max(rem, axis=1, keepdims=True)
        sel_idx = jnp.min(jnp.where(rem == m, lane, EPAD), axis=1, keepdims=True)
        idxs.append(sel_idx)
        vals.append(m)
        rem = jnp.where(lane == sel_idx, NEG, rem)

    exps = [jnp.exp(v - vals[0]) for v in vals]
    denom = exps[0] + exps[1] + exps[2] + exps[3]
    gates = jnp.zeros((N_TOK, EPAD), jnp.float32)
    for k in range(KTOP):
        gates = gates + jnp.where(lane == idxs[k], exps[k] / denom, 0.0)
    gates_ref[...] = gates

    emask = lane[0:1, :] < E
    importance = jnp.sum(gates, axis=0, keepdims=True)
    load = jnp.sum((gates > 0.0).astype(jnp.float32), axis=0, keepdims=True)

    def cv_sq(v):
        s = jnp.sum(jnp.where(emask, v, 0.0))
        mean = s / E
        var = jnp.sum(jnp.where(emask, (v - mean) ** 2, 0.0)) / (E - 1)
        return var / (mean * mean + 1e-10)

    loss_ref[0, 0] = (cv_sq(importance) + cv_sq(load)) * 1e-2


def _ffn_body(x_ref, gates_ref, fc1w_ref, fc1b_ref, fc2w_ref, fc2b_ref,
              y_ref, oe_acc, y_acc):
    e = pl.program_id(1)
    hc = pl.program_id(2)

    h = jnp.dot(x_ref[...], fc1w_ref[0], preferred_element_type=jnp.float32)
    h = jnp.maximum(h + fc1b_ref[0], 0.0)
    part = jnp.dot(h.astype(jnp.bfloat16), fc2w_ref[0],
                   preferred_element_type=jnp.float32)

    @pl.when(hc == 0)
    def _():
        oe_acc[...] = part + fc2b_ref[0]

    @pl.when(hc > 0)
    def _():
        oe_acc[...] += part

    @pl.when(hc == NHC - 1)
    def _():
        oe = oe_acc[...]
        m = jnp.max(oe, axis=1, keepdims=True)
        p = jnp.exp(oe - m)
        sm = p / jnp.sum(p, axis=1, keepdims=True)
        lane = jax.lax.broadcasted_iota(jnp.int32, (TOK_BLK, EPAD), 1)
        g = jnp.sum(jnp.where(lane == e, gates_ref[...], 0.0), axis=1,
                    keepdims=True)
        contrib = g * sm

        @pl.when(e == 0)
        def _():
            y_acc[...] = contrib

        @pl.when(e > 0)
        def _():
            y_acc[...] += contrib

        @pl.when(e == E - 1)
        def _():
            y_ref[...] = y_acc[...]


@jax.jit
def kernel(x, w_gate, fc1_w, fc1_b, fc2_w, fc2_b):
    b, l, d = x.shape
    xf = x.reshape(l, d)
    wg = jnp.zeros((D, EPAD), jnp.float32).at[:, :E].set(w_gate)

    gates, loss = pl.pallas_call(
        _routing_body,
        out_shape=(
            jax.ShapeDtypeStruct((N_TOK, EPAD), jnp.float32),
            jax.ShapeDtypeStruct((1, 1), jnp.float32),
        ),
        in_specs=[
            pl.BlockSpec((N_TOK, D), lambda: (0, 0)),
            pl.BlockSpec((D, EPAD), lambda: (0, 0)),
        ],
        out_specs=(
            pl.BlockSpec((N_TOK, EPAD), lambda: (0, 0)),
            pl.BlockSpec((1, 1), lambda: (0, 0), memory_space=pltpu.SMEM),
        ),
    )(xf, wg)

    y = pl.pallas_call(
        _ffn_body,
        grid=(NI, E, NHC),
        in_specs=[
            pl.BlockSpec((TOK_BLK, D), lambda i, e, hc: (i, 0)),
            pl.BlockSpec((TOK_BLK, EPAD), lambda i, e, hc: (i, 0)),
            pl.BlockSpec((1, D, H_BLK), lambda i, e, hc: (e, 0, hc)),
            pl.BlockSpec((1, 1, H_BLK), lambda i, e, hc: (e, 0, hc)),
            pl.BlockSpec((1, H_BLK, D), lambda i, e, hc: (e, hc, 0)),
            pl.BlockSpec((1, 1, D), lambda i, e, hc: (e, 0, 0)),
        ],
        out_specs=pl.BlockSpec((TOK_BLK, D), lambda i, e, hc: (i, 0)),
        out_shape=jax.ShapeDtypeStruct((N_TOK, D), jnp.float32),
        scratch_shapes=[
            pltpu.VMEM((TOK_BLK, D), jnp.float32),
            pltpu.VMEM((TOK_BLK, D), jnp.float32),
        ],
        compiler_params=pltpu.CompilerParams(
            dimension_semantics=("arbitrary", "arbitrary", "arbitrary"),
        ),
    )(xf.astype(jnp.bfloat16), gates, fc1_w.astype(jnp.bfloat16),
      fc1_b.reshape(E, 1, H), fc2_w.astype(jnp.bfloat16),
      fc2_b.reshape(E, 1, D))

    return y.reshape(b, l, d), loss.reshape(())
